# baseline (speedup 1.0000x reference)
import functools

import jax
import jax.numpy as jnp
from jax import lax
from jax.experimental import pallas as pl
from jax.experimental.pallas import tpu as pltpu
from jax.experimental.pallas import tpu_sc as plsc

B = 4096
L = 200
EMB = 64
VOCAB = 1000000
NC = 2
NS = 16
NW = NC * NS
RPW = B // NW
GROUPS = RPW // 16
C0, C1 = 104, 96
LP = 208
TBLK = 20000


def _matvec_body(w_ref, t_ref, o_ref):
    o_ref[...] = jnp.sum(t_ref[...] * w_ref[...], axis=1)[None, None, :]


def _matvec(W, table):
    nblk = VOCAB // TBLK
    out = pl.pallas_call(
        _matvec_body,
        grid=(nblk,),
        in_specs=[
            pl.BlockSpec((1, EMB), lambda i: (0, 0)),
            pl.BlockSpec((TBLK, EMB), lambda i: (i, 0)),
        ],
        out_specs=pl.BlockSpec((1, 1, TBLK), lambda i: (i, 0, 0)),
        out_shape=jax.ShapeDtypeStruct((nblk, 1, TBLK), jnp.float32),
    )(W, table)
    return out.reshape(-1)


_mesh = plsc.VectorSubcoreMesh(core_axis_name="c", subcore_axis_name="s")


@functools.partial(
    pl.kernel,
    out_type=jax.ShapeDtypeStruct((B,), jnp.float32),
    mesh=_mesh,
    compiler_params=pltpu.CompilerParams(
        needs_layout_passes=False, use_tc_tiling_on_sc=False),
    scratch_types=[
        pltpu.VMEM((RPW * L,), jnp.int32),
        pltpu.VMEM((2, 16 * LP), jnp.float32),
        pltpu.VMEM((RPW,), jnp.int32),
        pltpu.VMEM((16,), jnp.float32),
        pltpu.VMEM((RPW,), jnp.float32),
        pltpu.SemaphoreType.DMA,
    ],
)
def _pool_sc(data_hbm, len_hbm, s_hbm, b_hbm, out_hbm,
             idx_v, buf_v, len_v, b_v, out_v, sem):
    wid = lax.axis_index("s") * NC + lax.axis_index("c")
    base = wid * RPW

    pltpu.sync_copy(data_hbm.at[pl.ds(base * L, RPW * L)], idx_v)
    pltpu.sync_copy(len_hbm.at[pl.ds(base, RPW)], len_v)
    pltpu.sync_copy(b_hbm, b_v.at[pl.ds(0, 1)])

    zero = jnp.zeros((16,), jnp.float32)
    for slot in range(2):
        for j in range(16):
            buf_v[slot, pl.ds(j * LP + 192, 16)] = zero

    bias = b_v[pl.ds(0, 16)][0]
    lane = lax.iota(jnp.int32, 16)

    def fire_group(g, slot):
        for j in range(16):
            i = g * 16 + j
            pltpu.async_copy(
                s_hbm.at[idx_v.at[pl.ds(i * L, C0)]],
                buf_v.at[slot, pl.ds(j * LP, C0)], sem)
            pltpu.async_copy(
                s_hbm.at[idx_v.at[pl.ds(i * L + C0, C1)]],
                buf_v.at[slot, pl.ds(j * LP + C0, C1)], sem)

    fire_group(0, 0)

    def group_body(g, _):
        @pl.when(g < GROUPS - 1)
        def _():
            fire_group(g + 1, jnp.bitwise_and(g + 1, 1))

        slot = jnp.bitwise_and(g, 1)
        pltpu.make_async_copy(
            s_hbm.at[pl.ds(0, 16 * L)],
            buf_v.at[slot, pl.ds(0, 16 * L)], sem).wait()
        zvec = zero
        for j in range(16):
            p = zero
            for k in range(13):
                p = p + buf_v[slot, pl.ds(j * LP + k * 16, 16)]
            zvec = jnp.where(lane == j, jnp.sum(p), zvec)
        lvec = len_v[pl.ds(g * 16, 16)].astype(jnp.float32)
        zvec = zvec / lvec + bias
        out_v[pl.ds(g * 16, 16)] = 1.0 / (1.0 + jnp.exp(-zvec))
        return 0

    lax.fori_loop(0, GROUPS, group_body, 0)
    pltpu.sync_copy(out_v, out_hbm.at[pl.ds(base, RPW)])


def kernel(data, length, emb_table, W, b):
    s = _matvec(W, emb_table)
    return _pool_sc(data.reshape(-1), length, s, b)


# speedup vs baseline: 1.4488x; 1.4488x over previous
import functools

import jax
import jax.numpy as jnp
from jax import lax
from jax.experimental import pallas as pl
from jax.experimental.pallas import tpu as pltpu
from jax.experimental.pallas import tpu_sc as plsc

B = 4096
L = 200
EMB = 64
VOCAB = 1000000
NC = 2
NS = 16
NW = NC * NS
RPW = B // NW
GROUPS = RPW // 16
C0, C1 = 104, 96
LP = 208
TBLK = 50000


def _matvec_body(w_ref, t_ref, o_ref):
    o_ref[...] = jax.lax.dot_general(
        w_ref[...], t_ref[...], (((1,), (1,)), ((), ())),
        preferred_element_type=jnp.float32)[None]


def _matvec(W, table):
    nblk = VOCAB // TBLK
    out = pl.pallas_call(
        _matvec_body,
        grid=(nblk,),
        in_specs=[
            pl.BlockSpec((1, EMB), lambda i: (0, 0)),
            pl.BlockSpec((TBLK, EMB), lambda i: (i, 0)),
        ],
        out_specs=pl.BlockSpec((1, 1, TBLK), lambda i: (i, 0, 0)),
        out_shape=jax.ShapeDtypeStruct((nblk, 1, TBLK), jnp.float32),
    )(W, table)
    return out.reshape(-1)


_mesh = plsc.VectorSubcoreMesh(core_axis_name="c", subcore_axis_name="s")


@functools.partial(
    pl.kernel,
    out_type=jax.ShapeDtypeStruct((B,), jnp.float32),
    mesh=_mesh,
    compiler_params=pltpu.CompilerParams(
        needs_layout_passes=False, use_tc_tiling_on_sc=False),
    scratch_types=[
        pltpu.VMEM((RPW * L,), jnp.int32),
        pltpu.VMEM((2, 16 * LP), jnp.float32),
        pltpu.VMEM((RPW,), jnp.int32),
        pltpu.VMEM((16,), jnp.float32),
        pltpu.VMEM((RPW,), jnp.float32),
        pltpu.SemaphoreType.DMA,
    ],
)
def _pool_sc(data_hbm, len_hbm, s_hbm, b_hbm, out_hbm,
             idx_v, buf_v, len_v, b_v, out_v, sem):
    wid = lax.axis_index("s") * NC + lax.axis_index("c")
    base = wid * RPW

    pltpu.sync_copy(data_hbm.at[pl.ds(base * L, RPW * L)], idx_v)
    pltpu.sync_copy(len_hbm.at[pl.ds(base, RPW)], len_v)
    pltpu.sync_copy(b_hbm, b_v.at[pl.ds(0, 1)])

    zero = jnp.zeros((16,), jnp.float32)
    for slot in range(2):
        for j in range(16):
            buf_v[slot, pl.ds(j * LP + 192, 16)] = zero

    bias = b_v[pl.ds(0, 16)][0]
    lane = lax.iota(jnp.int32, 16)

    def fire_group(g, slot):
        for j in range(16):
            i = g * 16 + j
            pltpu.async_copy(
                s_hbm.at[idx_v.at[pl.ds(i * L, C0)]],
                buf_v.at[slot, pl.ds(j * LP, C0)], sem)
            pltpu.async_copy(
                s_hbm.at[idx_v.at[pl.ds(i * L + C0, C1)]],
                buf_v.at[slot, pl.ds(j * LP + C0, C1)], sem)

    fire_group(0, 0)

    def group_body(g, _):
        @pl.when(g < GROUPS - 1)
        def _():
            fire_group(g + 1, jnp.bitwise_and(g + 1, 1))

        slot = jnp.bitwise_and(g, 1)
        pltpu.make_async_copy(
            s_hbm.at[pl.ds(0, 16 * L)],
            buf_v.at[slot, pl.ds(0, 16 * L)], sem).wait()
        zvec = zero
        for j in range(16):
            p = zero
            for k in range(13):
                p = p + buf_v[slot, pl.ds(j * LP + k * 16, 16)]
            zvec = jnp.where(lane == j, jnp.sum(p), zvec)
        lvec = len_v[pl.ds(g * 16, 16)].astype(jnp.float32)
        zvec = zvec / lvec + bias
        out_v[pl.ds(g * 16, 16)] = 1.0 / (1.0 + jnp.exp(-zvec))
        return 0

    lax.fori_loop(0, GROUPS, group_body, 0)
    pltpu.sync_copy(out_v, out_hbm.at[pl.ds(base, RPW)])


def kernel(data, length, emb_table, W, b):
    s = _matvec(W, emb_table)
    return _pool_sc(data.reshape(-1), length, s, b)


# speedup vs baseline: 1.5069x; 1.0401x over previous
import functools

import jax
import jax.numpy as jnp
from jax import lax
from jax.experimental import pallas as pl
from jax.experimental.pallas import tpu as pltpu
from jax.experimental.pallas import tpu_sc as plsc

B = 4096
L = 200
EMB = 64
VOCAB = 1000000
NC = 2
NS = 16
NW = NC * NS
RPW = B // NW
GROUPS = RPW // 16
C0, C1 = 104, 96
LP = 208
TBLK = 25000


def _matvec_body(w_ref, ta_ref, tb_ref, oa_ref, ob_ref):
    dn = (((1,), (1,)), ((), ()))
    oa_ref[...] = jax.lax.dot_general(
        w_ref[...], ta_ref[...], dn,
        preferred_element_type=jnp.float32)[None]
    ob_ref[...] = jax.lax.dot_general(
        w_ref[...], tb_ref[...], dn,
        preferred_element_type=jnp.float32)[None]


def _matvec(W, table):
    nblk = VOCAB // TBLK
    half = nblk // 2
    oa, ob = pl.pallas_call(
        _matvec_body,
        grid=(half,),
        in_specs=[
            pl.BlockSpec((1, EMB), lambda i: (0, 0)),
            pl.BlockSpec((TBLK, EMB), lambda i: (i, 0)),
            pl.BlockSpec((TBLK, EMB), lambda i: (i + half, 0)),
        ],
        out_specs=[
            pl.BlockSpec((1, 1, TBLK), lambda i: (i, 0, 0)),
            pl.BlockSpec((1, 1, TBLK), lambda i: (i, 0, 0)),
        ],
        out_shape=[
            jax.ShapeDtypeStruct((half, 1, TBLK), jnp.float32),
            jax.ShapeDtypeStruct((half, 1, TBLK), jnp.float32),
        ],
    )(W, table, table)
    return jnp.concatenate([oa.reshape(-1), ob.reshape(-1)])


_mesh = plsc.VectorSubcoreMesh(core_axis_name="c", subcore_axis_name="s")


@functools.partial(
    pl.kernel,
    out_type=jax.ShapeDtypeStruct((B,), jnp.float32),
    mesh=_mesh,
    compiler_params=pltpu.CompilerParams(
        needs_layout_passes=False, use_tc_tiling_on_sc=False),
    scratch_types=[
        pltpu.VMEM((RPW * L,), jnp.int32),
        pltpu.VMEM((2, 16 * LP), jnp.float32),
        pltpu.VMEM((RPW,), jnp.int32),
        pltpu.VMEM((16,), jnp.float32),
        pltpu.VMEM((RPW,), jnp.float32),
        pltpu.SemaphoreType.DMA,
    ],
)
def _pool_sc(data_hbm, len_hbm, s_hbm, b_hbm, out_hbm,
             idx_v, buf_v, len_v, b_v, out_v, sem):
    wid = lax.axis_index("s") * NC + lax.axis_index("c")
    base = wid * RPW

    pltpu.sync_copy(data_hbm.at[pl.ds(base * L, RPW * L)], idx_v)
    pltpu.sync_copy(len_hbm.at[pl.ds(base, RPW)], len_v)
    pltpu.sync_copy(b_hbm, b_v.at[pl.ds(0, 1)])

    zero = jnp.zeros((16,), jnp.float32)
    for slot in range(2):
        for j in range(16):
            buf_v[slot, pl.ds(j * LP + 192, 16)] = zero

    bias = b_v[pl.ds(0, 16)][0]
    lane = lax.iota(jnp.int32, 16)

    def fire_group(g, slot):
        for j in range(16):
            i = g * 16 + j
            pltpu.async_copy(
                s_hbm.at[idx_v.at[pl.ds(i * L, C0)]],
                buf_v.at[slot, pl.ds(j * LP, C0)], sem)
            pltpu.async_copy(
                s_hbm.at[idx_v.at[pl.ds(i * L + C0, C1)]],
                buf_v.at[slot, pl.ds(j * LP + C0, C1)], sem)

    fire_group(0, 0)

    def group_body(g, _):
        @pl.when(g < GROUPS - 1)
        def _():
            fire_group(g + 1, jnp.bitwise_and(g + 1, 1))

        slot = jnp.bitwise_and(g, 1)
        pltpu.make_async_copy(
            s_hbm.at[pl.ds(0, 16 * L)],
            buf_v.at[slot, pl.ds(0, 16 * L)], sem).wait()
        zvec = zero
        for j in range(16):
            p = zero
            for k in range(13):
                p = p + buf_v[slot, pl.ds(j * LP + k * 16, 16)]
            zvec = jnp.where(lane == j, jnp.sum(p), zvec)
        lvec = len_v[pl.ds(g * 16, 16)].astype(jnp.float32)
        zvec = zvec / lvec + bias
        out_v[pl.ds(g * 16, 16)] = 1.0 / (1.0 + jnp.exp(-zvec))
        return 0

    lax.fori_loop(0, GROUPS, group_body, 0)
    pltpu.sync_copy(out_v, out_hbm.at[pl.ds(base, RPW)])


def kernel(data, length, emb_table, W, b):
    s = _matvec(W, emb_table)
    return _pool_sc(data.reshape(-1), length, s, b)
